# baseline (speedup 1.0000x reference)
import functools

import jax
import jax.numpy as jnp
from jax import lax
from jax.experimental import pallas as pl
from jax.experimental.pallas import tpu as pltpu
from jax.experimental.pallas import tpu_sc as plsc

N_NODES = 100000
N_EDGES = 1600000
EMB = 32

C = 400
SUB = 100
NSUB = C // SUB
NGROUP = C // 16
NW = 32
NCHUNKS = N_EDGES // C
NPW = NCHUNKS // NW

_MAGIC = 0x5F3759DF


def _sqrt16(d2):
    xc = jnp.maximum(d2, jnp.float32(1e-30))
    ii = plsc.bitcast(xc, jnp.int32)
    ii = jnp.int32(_MAGIC) - lax.shift_right_logical(ii, 1)
    y = plsc.bitcast(ii, jnp.float32)
    xh = xc * jnp.float32(0.5)
    y = y * (jnp.float32(1.5) - xh * y * y)
    y = y * (jnp.float32(1.5) - xh * y * y)
    y = y * (jnp.float32(1.5) - xh * y * y)
    return jnp.where(d2 > jnp.float32(1e-30), xc * y, jnp.float32(0.0))


def _make_edge_loss():
    mesh = plsc.VectorSubcoreMesh(core_axis_name="c", subcore_axis_name="s")

    @functools.partial(
        pl.kernel,
        mesh=mesh,
        compiler_params=pltpu.CompilerParams(
            needs_layout_passes=False, use_tc_tiling_on_sc=False),
        out_type=jax.ShapeDtypeStruct((NW, 16), jnp.float32),
        scratch_types=[
            pltpu.VMEM((2, NSUB, SUB), jnp.int32),
            pltpu.VMEM((2, NSUB, SUB), jnp.int32),
            pltpu.VMEM((2, C), jnp.float32),
            pltpu.VMEM((2, C, EMB), jnp.float32),
            pltpu.VMEM((2, C, EMB), jnp.float32),
            pltpu.VMEM((16,), jnp.float32),
            pltpu.SemaphoreType.DMA((2,)),
            pltpu.SemaphoreType.DMA((2,)),
        ],
    )
    def edge_loss(y_hbm, row_hbm, col_hbm, w_hbm, out_hbm,
                  ridx, cidx, wv, va, vb, accv, sem_i, sem_g):
        cid = lax.axis_index("c")
        sid = lax.axis_index("s")
        wid = sid * 2 + cid
        base = wid * NPW

        def idx_copies(c, b):
            return [
                pltpu.make_async_copy(
                    row_hbm.at[pl.ds(c * NSUB, NSUB)], ridx.at[b], sem_i.at[b]),
                pltpu.make_async_copy(
                    col_hbm.at[pl.ds(c * NSUB, NSUB)], cidx.at[b], sem_i.at[b]),
            ]

        def gather_copies(c, b):
            cps = []
            for j in range(NSUB):
                cps.append(pltpu.make_async_copy(
                    y_hbm.at[ridx.at[b, j]],
                    va.at[b, pl.ds(j * SUB, SUB)], sem_g.at[b]))
                cps.append(pltpu.make_async_copy(
                    y_hbm.at[cidx.at[b, j]],
                    vb.at[b, pl.ds(j * SUB, SUB)], sem_g.at[b]))
            cps.append(pltpu.make_async_copy(
                w_hbm.at[pl.ds(c * C, C)], wv.at[b], sem_g.at[b]))
            return cps

        def start(cps):
            for cp in cps:
                cp.start()

        def wait(cps):
            for cp in cps:
                cp.wait()

        def compute(b, acc):
            bfull = jnp.full((16,), 0, jnp.int32) + b
            lane = lax.iota(jnp.int32, 16)

            def dist_w(g):
                eids = g * 16 + lane
                d2a = jnp.zeros((16,), jnp.float32)
                d2b = jnp.zeros((16,), jnp.float32)
                for k in range(EMB):
                    ck = (lane + k) & (EMB - 1)
                    a = plsc.load_gather(va, [bfull, eids, ck])
                    bk = plsc.load_gather(vb, [bfull, eids, ck])
                    d = a - bk
                    if k % 2 == 0:
                        d2a = d2a + d * d
                    else:
                        d2b = d2b + d * d
                dist = _sqrt16(d2a + d2b)
                return dist * wv[b, pl.ds(g * 16, 16)]

            def group_body(g2, acc2):
                return acc2 + dist_w(2 * g2) + dist_w(2 * g2 + 1)

            acc = lax.fori_loop(0, NGROUP // 2, group_body, acc)
            if NGROUP % 2:
                acc = acc + dist_w(NGROUP - 1)
            return acc

        start(idx_copies(base, 0))
        wait(idx_copies(base, 0))
        start(gather_copies(base, 0))
        start(idx_copies(base + 1, 1))

        def chunk_body(i, acc):
            c = base + i
            b = lax.rem(i, 2)
            nb = 1 - b
            wait(gather_copies(c, b))
            wait(idx_copies(c + 1, nb))
            start(gather_copies(c + 1, nb))
            start(idx_copies(c + 2, b))
            return compute(b, acc)

        acc = lax.fori_loop(0, NPW - 2, chunk_body,
                            jnp.zeros((16,), jnp.float32))

        b2 = (NPW - 2) % 2
        b1 = (NPW - 1) % 2
        c2 = base + NPW - 2
        c1 = base + NPW - 1
        wait(gather_copies(c2, b2))
        wait(idx_copies(c1, b1))
        start(gather_copies(c1, b1))
        acc = compute(b2, acc)
        wait(gather_copies(c1, b1))
        acc = compute(b1, acc)

        accv[...] = acc
        pltpu.sync_copy(accv, out_hbm.at[wid])

    return edge_loss


_edge_loss = _make_edge_loss()


def kernel(Y, edge_index, edge_weight):
    row = edge_index[0].astype(jnp.int32).reshape(NCHUNKS * NSUB, SUB)
    col = edge_index[1].astype(jnp.int32).reshape(NCHUNKS * NSUB, SUB)
    partial = _edge_loss(Y, row, col, edge_weight)
    return jnp.sum(partial) / jnp.float32(N_EDGES)


# speedup vs baseline: 1.5200x; 1.5200x over previous
import functools

import jax
import jax.numpy as jnp
from jax import lax
from jax.experimental import pallas as pl
from jax.experimental.pallas import tpu as pltpu
from jax.experimental.pallas import tpu_sc as plsc

N_NODES = 100000
N_EDGES = 1600000
EMB = 32

C = 400
SUB = 100
NSUB = C // SUB
NGROUP = C // 16
NW = 32
NCHUNKS = N_EDGES // C
NPW = NCHUNKS // NW

_MAGIC = 0x5F3759DF


def _sqrt16(d2):
    xc = jnp.maximum(d2, jnp.float32(1e-30))
    ii = plsc.bitcast(xc, jnp.int32)
    ii = jnp.int32(_MAGIC) - lax.shift_right_logical(ii, 1)
    y = plsc.bitcast(ii, jnp.float32)
    xh = xc * jnp.float32(0.5)
    y = y * (jnp.float32(1.5) - xh * y * y)
    y = y * (jnp.float32(1.5) - xh * y * y)
    y = y * (jnp.float32(1.5) - xh * y * y)
    return jnp.where(d2 > jnp.float32(1e-30), xc * y, jnp.float32(0.0))


def _make_edge_loss():
    mesh = plsc.VectorSubcoreMesh(core_axis_name="c", subcore_axis_name="s")

    @functools.partial(
        pl.kernel,
        mesh=mesh,
        compiler_params=pltpu.CompilerParams(
            needs_layout_passes=False, use_tc_tiling_on_sc=False),
        out_type=jax.ShapeDtypeStruct((NW, 16), jnp.float32),
        scratch_types=[
            pltpu.VMEM((2, NSUB, SUB), jnp.int32),
            pltpu.VMEM((2, NSUB, SUB), jnp.int32),
            pltpu.VMEM((2, C), jnp.float32),
            pltpu.VMEM((2, C, EMB), jnp.float32),
            pltpu.VMEM((2, C, EMB), jnp.float32),
            pltpu.VMEM((16,), jnp.float32),
            pltpu.SemaphoreType.DMA((2,)),
            pltpu.SemaphoreType.DMA((2,)),
        ],
    )
    def edge_loss(y_hbm, row_hbm, col_hbm, w_hbm, out_hbm,
                  ridx, cidx, wv, va, vb, accv, sem_i, sem_g):
        cid = lax.axis_index("c")
        sid = lax.axis_index("s")
        wid = sid * 2 + cid
        base = wid * NPW

        def idx_copies(c, b):
            return [
                pltpu.make_async_copy(
                    row_hbm.at[pl.ds(c * NSUB, NSUB)], ridx.at[b], sem_i.at[b]),
                pltpu.make_async_copy(
                    col_hbm.at[pl.ds(c * NSUB, NSUB)], cidx.at[b], sem_i.at[b]),
            ]

        def gather_copies(c, b):
            cps = []
            for j in range(NSUB):
                cps.append(pltpu.make_async_copy(
                    y_hbm.at[ridx.at[b, j]],
                    va.at[b, pl.ds(j * SUB, SUB)], sem_g.at[b]))
                cps.append(pltpu.make_async_copy(
                    y_hbm.at[cidx.at[b, j]],
                    vb.at[b, pl.ds(j * SUB, SUB)], sem_g.at[b]))
            cps.append(pltpu.make_async_copy(
                w_hbm.at[pl.ds(c * C, C)], wv.at[b], sem_g.at[b]))
            return cps

        def start(cps):
            for cp in cps:
                cp.start()

        def wait(cps):
            for cp in cps:
                cp.wait()

        def compute(b, acc):
            bfull = jnp.full((16,), 0, jnp.int32) + b
            lane = lax.iota(jnp.int32, 16)

            def group_body(g, acc2):
                eids = g * 16 + lane
                d2 = jnp.zeros((16,), jnp.float32)
                for k in range(EMB):
                    ck = (lane + k) & (EMB - 1)
                    a = plsc.load_gather(va, [bfull, eids, ck])
                    bk = plsc.load_gather(vb, [bfull, eids, ck])
                    d = a - bk
                    d2 = d2 + d * d
                dist = _sqrt16(d2)
                return acc2 + dist * wv[b, pl.ds(g * 16, 16)]

            return lax.fori_loop(0, NGROUP, group_body, acc)

        start(idx_copies(base, 0))
        wait(idx_copies(base, 0))
        start(gather_copies(base, 0))
        start(idx_copies(base + 1, 1))

        def chunk_body(i, acc):
            c = base + i
            b = lax.rem(i, 2)
            nb = 1 - b
            wait(gather_copies(c, b))
            wait(idx_copies(c + 1, nb))
            start(gather_copies(c + 1, nb))
            start(idx_copies(c + 2, b))
            return compute(b, acc)

        acc = lax.fori_loop(0, NPW - 2, chunk_body,
                            jnp.zeros((16,), jnp.float32))

        b2 = (NPW - 2) % 2
        b1 = (NPW - 1) % 2
        c2 = base + NPW - 2
        c1 = base + NPW - 1
        wait(gather_copies(c2, b2))
        wait(idx_copies(c1, b1))
        start(gather_copies(c1, b1))
        acc = compute(b2, acc)
        wait(gather_copies(c1, b1))
        acc = compute(b1, acc)

        accv[...] = acc
        pltpu.sync_copy(accv, out_hbm.at[wid])

    return edge_loss


_edge_loss = _make_edge_loss()


def kernel(Y, edge_index, edge_weight):
    row = edge_index[0].astype(jnp.int32).reshape(NCHUNKS * NSUB, SUB)
    col = edge_index[1].astype(jnp.int32).reshape(NCHUNKS * NSUB, SUB)
    partial = _edge_loss(Y, row, col, edge_weight)
    return jnp.sum(partial) / jnp.float32(N_EDGES)
